# baseline (speedup 1.0000x reference)
import functools

import jax
import jax.numpy as jnp
from jax import lax
from jax.experimental import pallas as pl
from jax.experimental.pallas import tpu as pltpu
from jax.experimental.pallas import tpu_sc as plsc

DIM = 16
B = 16384
L = 20
LANES = 16

NC = 2
NS = 16
NW = NC * NS

SEQ_PER_W = B // NW
IDX_PER_W = SEQ_PER_W * L
CHUNK_SEQS = 64
CHUNK_IDX = CHUNK_SEQS * L
N_CHUNKS = SEQ_PER_W // CHUNK_SEQS
G_ROWS = 128
G_PER_CHUNK = CHUNK_IDX // G_ROWS
ROWS_PER_W = IDX_PER_W // G_ROWS
IDX_PAD = 16


def _body(seq_flat_hbm, table_hbm, out_hbm,
          idx_flat, rows0, rows1, outbuf, t0_v,
          sem0, sem1):
    wid = lax.axis_index("s") * NC + lax.axis_index("c")

    pltpu.sync_copy(seq_flat_hbm.at[pl.ds(wid * IDX_PER_W, IDX_PER_W)],
                    idx_flat.at[pl.ds(0, IDX_PER_W)])
    pltpu.sync_copy(table_hbm.at[pl.ds(0, 1)], t0_v)

    rows_bufs = (rows0, rows1)
    sems = (sem0, sem1)

    def fire(chunk):
        buf = rows_bufs[chunk % 2]
        sem = sems[chunk % 2]
        cps = []
        for g in range(G_PER_CHUNK):
            idx_row = idx_flat.at[
                pl.ds((chunk * G_PER_CHUNK + g) * G_ROWS, G_ROWS)]
            dst = buf.at[pl.ds(g * G_ROWS, G_ROWS)]
            cps.append(pltpu.async_copy(table_hbm.at[idx_row], dst, sem))
        return cps

    inflight = [fire(0), fire(1)]

    lane_iota = lax.iota(jnp.int32, LANES)
    lo4 = jnp.where(lane_iota < 4, 1.0, 0.0).astype(jnp.float32)
    lo8 = jnp.where(lane_iota < 8, 1.0, 0.0).astype(jnp.float32)
    t0 = t0_v[0]
    lf = jnp.float32(L)
    one = jnp.float32(1.0)

    for chunk in range(N_CHUNKS):
        buf = rows_bufs[chunk % 2]
        for cp in inflight[chunk % 2]:
            cp.wait()

        def pair_body(p, _, chunk=chunk, buf=buf):
            rb = p * (2 * L)
            acc0 = buf[rb]
            for j in range(1, L):
                acc0 = acc0 + buf[rb + j]
            acc1 = buf[rb + L]
            for j in range(L + 1, 2 * L):
                acc1 = acc1 + buf[rb + j]

            ib = chunk * CHUNK_IDX + rb
            v0 = idx_flat[pl.ds(ib, LANES)]
            v1 = idx_flat[pl.ds(ib + LANES, LANES)]
            v2 = idx_flat[pl.ds(ib + 2 * LANES, LANES)]
            z0 = jnp.where(v0 == 0, 1.0, 0.0).astype(jnp.float32)
            z1 = jnp.where(v1 == 0, 1.0, 0.0).astype(jnp.float32)
            z2 = jnp.where(v2 == 0, 1.0, 0.0).astype(jnp.float32)
            za = jnp.full((LANES,), jnp.sum(z0) + jnp.sum(z1 * lo4))
            zb = jnp.full((LANES,),
                          jnp.sum(z1 * (one - lo4)) + jnp.sum(z2 * lo8))
            s0 = 1.0 / jnp.maximum(lf - za, 1.0)
            s1 = 1.0 / jnp.maximum(lf - zb, 1.0)

            sgw = chunk * CHUNK_SEQS + 2 * p
            outbuf[sgw] = (acc0 - za * t0) * s0
            outbuf[sgw + 1] = (acc1 - zb * t0) * s1
            return _

        lax.fori_loop(0, CHUNK_SEQS // 2, pair_body, 0, unroll=False)

        if chunk + 2 < N_CHUNKS:
            inflight[chunk % 2] = fire(chunk + 2)

    pltpu.sync_copy(outbuf, out_hbm.at[pl.ds(wid * SEQ_PER_W, SEQ_PER_W)])


TBLK = 8192


def _transpose_body(t_ref, o_ref):
    eye = jnp.eye(16, dtype=jnp.float32)
    o_ref[...] = jax.lax.dot_general(
        t_ref[...], eye, (((0,), (0,)), ((), ())),
        preferred_element_type=jnp.float32)


def _tc_transpose(table_t):
    vocab = table_t.shape[1]
    grid = (vocab + TBLK - 1) // TBLK
    return pl.pallas_call(
        _transpose_body,
        grid=(grid,),
        in_specs=[pl.BlockSpec((16, TBLK), lambda i: (0, i))],
        out_specs=pl.BlockSpec((TBLK, 16), lambda i: (i, 0)),
        out_shape=jax.ShapeDtypeStruct((vocab, 16), jnp.float32),
    )(table_t)


@jax.jit
def _run(seq_flat, table):
    mesh = plsc.VectorSubcoreMesh(core_axis_name="c", subcore_axis_name="s")
    k = functools.partial(
        pl.kernel,
        mesh=mesh,
        out_type=jax.ShapeDtypeStruct((B, DIM), jnp.float32),
        compiler_params=pltpu.CompilerParams(
            needs_layout_passes=False, use_tc_tiling_on_sc=False),
        scratch_types=[
            pltpu.VMEM((IDX_PER_W + IDX_PAD,), jnp.int32),
            pltpu.VMEM((CHUNK_IDX, DIM), jnp.float32),
            pltpu.VMEM((CHUNK_IDX, DIM), jnp.float32),
            pltpu.VMEM((SEQ_PER_W, DIM), jnp.float32),
            pltpu.VMEM((1, DIM), jnp.float32),
            pltpu.SemaphoreType.DMA,
            pltpu.SemaphoreType.DMA,
        ],
    )(_body)
    table_rm = _tc_transpose(table.T)
    return k(seq_flat, table_rm)


def kernel(sequences, table):
    return _run(sequences.reshape(B * L), table)


# speedup vs baseline: 4.9190x; 4.9190x over previous
import functools

import jax
import jax.numpy as jnp
from jax import lax
from jax.experimental import pallas as pl
from jax.experimental.pallas import tpu as pltpu
from jax.experimental.pallas import tpu_sc as plsc

VOCAB = 1000000
DIM = 16
B = 16384
L = 20
LANES = 16

NC = 2
NS = 16
NW = NC * NS

SEQ_PER_W = B // NW
IDX_PER_W = SEQ_PER_W * L
CHUNK_SEQS = 64
CHUNK_IDX = CHUNK_SEQS * L
N_CHUNKS = SEQ_PER_W // CHUNK_SEQS
G_ROWS = 128
G_PER_CHUNK = CHUNK_IDX // G_ROWS
ROWS_PER_W = IDX_PER_W // G_ROWS
IDX_PAD = 16


def _body(seq_flat_hbm, table_hbm, out_hbm,
          idx_flat, rows0, rows1, outbuf, t0_v,
          sem0, sem1):
    wid = lax.axis_index("s") * NC + lax.axis_index("c")

    pltpu.sync_copy(seq_flat_hbm.at[pl.ds(wid * IDX_PER_W, IDX_PER_W)],
                    idx_flat.at[pl.ds(0, IDX_PER_W)])
    pltpu.sync_copy(table_hbm.at[pl.ds(0, 1)], t0_v)

    def xform(i, carry):
        v = idx_flat[pl.ds(i * LANES, LANES)]
        idx_flat[pl.ds(i * LANES, LANES)] = (
            ((v & (SLOT_W - 1)) << 3) | (v >> SLOT_BITS))
        return carry

    lax.fori_loop(0, IDX_PER_W // LANES, xform, 0, unroll=False)

    rows_bufs = (rows0, rows1)
    sems = (sem0, sem1)

    def fire(chunk):
        buf = rows_bufs[chunk % 2]
        sem = sems[chunk % 2]
        cps = []
        for g in range(G_PER_CHUNK):
            idx_row = idx_flat.at[
                pl.ds((chunk * G_PER_CHUNK + g) * G_ROWS, G_ROWS)]
            dst = buf.at[pl.ds(g * G_ROWS, G_ROWS)]
            cps.append(pltpu.async_copy(table_hbm.at[idx_row], dst, sem))
        return cps

    inflight = [fire(0), fire(1)]

    lane_iota = lax.iota(jnp.int32, LANES)
    lo4 = jnp.where(lane_iota < 4, 1.0, 0.0).astype(jnp.float32)
    lo8 = jnp.where(lane_iota < 8, 1.0, 0.0).astype(jnp.float32)
    t0 = t0_v[0]
    lf = jnp.float32(L)
    one = jnp.float32(1.0)

    for chunk in range(N_CHUNKS):
        buf = rows_bufs[chunk % 2]
        for cp in inflight[chunk % 2]:
            cp.wait()

        def pair_body(p, _, chunk=chunk, buf=buf):
            rb = p * (2 * L)
            acc0 = buf[rb]
            for j in range(1, L):
                acc0 = acc0 + buf[rb + j]
            acc1 = buf[rb + L]
            for j in range(L + 1, 2 * L):
                acc1 = acc1 + buf[rb + j]

            ib = chunk * CHUNK_IDX + rb
            v0 = idx_flat[pl.ds(ib, LANES)]
            v1 = idx_flat[pl.ds(ib + LANES, LANES)]
            v2 = idx_flat[pl.ds(ib + 2 * LANES, LANES)]
            z0 = jnp.where(v0 == 0, 1.0, 0.0).astype(jnp.float32)
            z1 = jnp.where(v1 == 0, 1.0, 0.0).astype(jnp.float32)
            z2 = jnp.where(v2 == 0, 1.0, 0.0).astype(jnp.float32)
            za = jnp.full((LANES,), jnp.sum(z0) + jnp.sum(z1 * lo4))
            zb = jnp.full((LANES,),
                          jnp.sum(z1 * (one - lo4)) + jnp.sum(z2 * lo8))
            s0 = 1.0 / jnp.maximum(lf - za, 1.0)
            s1 = 1.0 / jnp.maximum(lf - zb, 1.0)

            sgw = chunk * CHUNK_SEQS + 2 * p
            outbuf[sgw] = (acc0 - za * t0) * s0
            outbuf[sgw + 1] = (acc1 - zb * t0) * s1
            return _

        lax.fori_loop(0, CHUNK_SEQS // 2, pair_body, 0, unroll=False)

        if chunk + 2 < N_CHUNKS:
            inflight[chunk % 2] = fire(chunk + 2)

    pltpu.sync_copy(outbuf, out_hbm.at[pl.ds(wid * SEQ_PER_W, SEQ_PER_W)])


SLOT_BITS = 17
SLOT_W = 1 << SLOT_BITS
SLOTS = 8
TBLK = 4096
RGRID = SLOT_W // TBLK
_LAST_IN_BLK = (VOCAB + TBLK - 1) // TBLK - 1


def _repack_body(*refs):
    o_ref = refs[SLOTS]
    g = jnp.concatenate([r[...] for r in refs[:SLOTS]], axis=0)
    o_ref[...] = g.T


def _tc_repack(table_t):
    def mk(s):
        return pl.BlockSpec(
            (16, TBLK), lambda i, s=s: (0, jnp.minimum(s * RGRID + i,
                                                       _LAST_IN_BLK)))
    return pl.pallas_call(
        _repack_body,
        grid=(RGRID,),
        in_specs=[mk(s) for s in range(SLOTS)],
        out_specs=pl.BlockSpec((TBLK, 128), lambda i: (i, 0)),
        out_shape=jax.ShapeDtypeStruct((SLOT_W, 128), jnp.float32),
    )(*([table_t] * SLOTS))


@jax.jit
def _run(seq_flat, table):
    mesh = plsc.VectorSubcoreMesh(core_axis_name="c", subcore_axis_name="s")
    k = functools.partial(
        pl.kernel,
        mesh=mesh,
        out_type=jax.ShapeDtypeStruct((B, DIM), jnp.float32),
        compiler_params=pltpu.CompilerParams(
            needs_layout_passes=False, use_tc_tiling_on_sc=False),
        scratch_types=[
            pltpu.VMEM((IDX_PER_W + IDX_PAD,), jnp.int32),
            pltpu.VMEM((CHUNK_IDX, DIM), jnp.float32),
            pltpu.VMEM((CHUNK_IDX, DIM), jnp.float32),
            pltpu.VMEM((SEQ_PER_W, DIM), jnp.float32),
            pltpu.VMEM((1, DIM), jnp.float32),
            pltpu.SemaphoreType.DMA,
            pltpu.SemaphoreType.DMA,
        ],
    )(_body)
    table_rm = _tc_repack(table.T).reshape(SLOTS * SLOT_W, DIM)
    return k(seq_flat, table_rm)


def kernel(sequences, table):
    return _run(sequences.reshape(B * L), table)


# speedup vs baseline: 5.9922x; 1.2182x over previous
import functools

import jax
import jax.numpy as jnp
from jax import lax
from jax.experimental import pallas as pl
from jax.experimental.pallas import tpu as pltpu
from jax.experimental.pallas import tpu_sc as plsc

VOCAB = 1000000
DIM = 16
B = 16384
L = 20
LANES = 16

NC = 2
NS = 16
NW = NC * NS

SEQ_PER_W = B // NW
IDX_PER_W = SEQ_PER_W * L
CHUNK_SEQS = 64
CHUNK_IDX = CHUNK_SEQS * L
N_CHUNKS = SEQ_PER_W // CHUNK_SEQS

XQ = B // 8


def _body(seq_cm_hbm, table_hbm, out_hbm,
          idx_flat, rows0, rows1, outbuf, cnt_v, t0_v,
          sem0, sem1, sem2):
    wid = lax.axis_index("s") * NC + lax.axis_index("c")
    sbase = wid * SEQ_PER_W

    stage = []
    for t in range(L):
        stage.append(pltpu.async_copy(
            seq_cm_hbm.at[pl.ds(t * B + sbase, SEQ_PER_W)],
            idx_flat.at[pl.ds(t * SEQ_PER_W, SEQ_PER_W)], sem2))
    pltpu.sync_copy(table_hbm.at[pl.ds(0, 1)], t0_v)
    for cp in stage:
        cp.wait()

    def xform(i, carry):
        v = idx_flat[pl.ds(i * LANES, LANES)]
        idx_flat[pl.ds(i * LANES, LANES)] = (
            ((v & (SLOT_W - 1)) << 3) | (v >> SLOT_BITS))
        return carry

    lax.fori_loop(0, IDX_PER_W // LANES, xform, 0, unroll=False)

    rows_bufs = (rows0, rows1)
    sems = (sem0, sem1)

    def fire(chunk):
        buf = rows_bufs[chunk % 2]
        sem = sems[chunk % 2]
        cps = []
        for t in range(L):
            idx_row = idx_flat.at[
                pl.ds(t * SEQ_PER_W + chunk * CHUNK_SEQS, CHUNK_SEQS)]
            dst = buf.at[pl.ds(t * CHUNK_SEQS, CHUNK_SEQS)]
            cps.append(pltpu.async_copy(table_hbm.at[idx_row], dst, sem))
        return cps

    inflight = [fire(0), fire(1)]

    t0 = t0_v[0]
    lf = jnp.float32(L)
    lane_iota = lax.iota(jnp.int32, LANES)

    for chunk in range(N_CHUNKS):
        buf = rows_bufs[chunk % 2]
        for cp in inflight[chunk % 2]:
            cp.wait()

        for g in range(CHUNK_SEQS // LANES):
            cnt = jnp.zeros((LANES,), jnp.float32)
            for t in range(L):
                v = idx_flat[pl.ds(
                    t * SEQ_PER_W + chunk * CHUNK_SEQS + g * LANES, LANES)]
                cnt = cnt + jnp.where(v == 0, 1.0, 0.0).astype(jnp.float32)
            cnt_v[pl.ds(g * LANES, LANES)] = cnt

        def seq_body(i, carry, chunk=chunk, buf=buf):
            acc = buf[i]
            for t in range(1, L):
                acc = acc + buf[t * CHUNK_SEQS + i]
            gbase = (i // LANES) * LANES
            cnt16 = cnt_v[pl.ds(gbase, LANES)]
            sel = jnp.where(lane_iota == i - gbase, 1.0, 0.0)
            za = jnp.full((LANES,), jnp.sum(cnt16 * sel.astype(jnp.float32)))
            inv = 1.0 / jnp.maximum(lf - za, 1.0)
            outbuf[chunk * CHUNK_SEQS + i] = (acc - za * t0) * inv
            return carry

        lax.fori_loop(0, CHUNK_SEQS, seq_body, 0, unroll=False)

        if chunk + 2 < N_CHUNKS:
            inflight[chunk % 2] = fire(chunk + 2)

    pltpu.sync_copy(
        outbuf,
        out_hbm.at[pl.ds((wid % 4) * SEQ_PER_W, SEQ_PER_W),
                   pl.ds((wid // 4) * DIM, DIM)])


SLOT_BITS = 17
SLOT_W = 1 << SLOT_BITS
SLOTS = 8
TBLK = 4096
RGRID = SLOT_W // TBLK
_LAST_IN_BLK = (VOCAB + TBLK - 1) // TBLK - 1


def _repack_body(*refs):
    o_ref = refs[SLOTS]
    g = jnp.concatenate([r[...] for r in refs[:SLOTS]], axis=0)
    o_ref[...] = g.T


def _tc_repack(table_t):
    def mk(s):
        return pl.BlockSpec(
            (16, TBLK), lambda i, s=s: (0, jnp.minimum(s * RGRID + i,
                                                       _LAST_IN_BLK)))
    return pl.pallas_call(
        _repack_body,
        grid=(RGRID,),
        in_specs=[mk(s) for s in range(SLOTS)],
        out_specs=pl.BlockSpec((TBLK, 128), lambda i: (i, 0)),
        out_shape=jax.ShapeDtypeStruct((SLOT_W, 128), jnp.float32),
    )(*([table_t] * SLOTS))


def _unpack_body(x_ref, o_ref):
    y = x_ref[...].T
    for g in range(8):
        o_ref[:, g * XQ:(g + 1) * XQ] = y[g * DIM:(g + 1) * DIM, :]


def _tc_unpack(x):
    return pl.pallas_call(
        _unpack_body,
        out_shape=jax.ShapeDtypeStruct((DIM, B), jnp.float32),
    )(x)


@jax.jit
def _run(seq_cm, table):
    mesh = plsc.VectorSubcoreMesh(core_axis_name="c", subcore_axis_name="s")
    k = functools.partial(
        pl.kernel,
        mesh=mesh,
        out_type=jax.ShapeDtypeStruct((XQ, 128), jnp.float32),
        compiler_params=pltpu.CompilerParams(
            needs_layout_passes=False, use_tc_tiling_on_sc=False),
        scratch_types=[
            pltpu.VMEM((IDX_PER_W,), jnp.int32),
            pltpu.VMEM((CHUNK_IDX, DIM), jnp.float32),
            pltpu.VMEM((CHUNK_IDX, DIM), jnp.float32),
            pltpu.VMEM((SEQ_PER_W, DIM), jnp.float32),
            pltpu.VMEM((CHUNK_SEQS,), jnp.float32),
            pltpu.VMEM((1, DIM), jnp.float32),
            pltpu.SemaphoreType.DMA,
            pltpu.SemaphoreType.DMA,
            pltpu.SemaphoreType.DMA,
        ],
    )(_body)
    table_rm = _tc_repack(table.T).reshape(SLOTS * SLOT_W, DIM)
    return _tc_unpack(k(seq_cm, table_rm)).T


def kernel(sequences, table):
    return _run(sequences.T.reshape(B * L), table)


# speedup vs baseline: 6.4559x; 1.0774x over previous
import functools

import jax
import jax.numpy as jnp
from jax import lax
from jax.experimental import pallas as pl
from jax.experimental.pallas import tpu as pltpu
from jax.experimental.pallas import tpu_sc as plsc

VOCAB = 1000000
DIM = 16
B = 16384
L = 20
LANES = 16

NC = 2
NS = 16
NW = NC * NS

SEQ_PER_W = B // NW
IDX_PER_W = SEQ_PER_W * L
CHUNK_SEQS = 64
CHUNK_IDX = CHUNK_SEQS * L
N_CHUNKS = SEQ_PER_W // CHUNK_SEQS

XQ = B // 8


def _body(seq_cm_hbm, table_hbm, out_hbm,
          idx_flat, rows0, rows1, outbuf, cnt_v, t0_v,
          sem0, sem1, sem2):
    wid = lax.axis_index("s") * NC + lax.axis_index("c")
    sbase = wid * SEQ_PER_W

    stage = []
    for t in range(L):
        stage.append(pltpu.async_copy(
            seq_cm_hbm.at[pl.ds(t * B + sbase, SEQ_PER_W)],
            idx_flat.at[pl.ds(t * SEQ_PER_W, SEQ_PER_W)], sem2))
    pltpu.sync_copy(table_hbm.at[pl.ds(0, 1)], t0_v)
    for cp in stage:
        cp.wait()

    def xform(i, carry):
        v = idx_flat[pl.ds(i * LANES, LANES)]
        idx_flat[pl.ds(i * LANES, LANES)] = (
            ((v & (SLOT_W - 1)) << 3) | (v >> SLOT_BITS))
        return carry

    lax.fori_loop(0, IDX_PER_W // LANES, xform, 0, unroll=False)

    rows_bufs = (rows0, rows1)
    sems = (sem0, sem1)

    def fire(chunk):
        buf = rows_bufs[chunk % 2]
        sem = sems[chunk % 2]
        cps = []
        for t in range(L):
            idx_row = idx_flat.at[
                pl.ds(t * SEQ_PER_W + chunk * CHUNK_SEQS, CHUNK_SEQS)]
            dst = buf.at[pl.ds(t * CHUNK_SEQS, CHUNK_SEQS)]
            cps.append(pltpu.async_copy(table_hbm.at[idx_row], dst, sem))
        return cps

    inflight = [fire(0), fire(1)]

    t0 = t0_v[0]
    lf = jnp.float32(L)
    lane_iota = lax.iota(jnp.int32, LANES)

    for chunk in range(N_CHUNKS):
        buf = rows_bufs[chunk % 2]
        for cp in inflight[chunk % 2]:
            cp.wait()

        for g in range(CHUNK_SEQS // LANES):
            cnt = jnp.zeros((LANES,), jnp.float32)
            for t in range(L):
                v = idx_flat[pl.ds(
                    t * SEQ_PER_W + chunk * CHUNK_SEQS + g * LANES, LANES)]
                cnt = cnt + jnp.where(v == 0, 1.0, 0.0).astype(jnp.float32)
            cnt_v[pl.ds(g * LANES, LANES)] = cnt

        def seq_body(i, carry, chunk=chunk, buf=buf):
            acc = buf[i]
            for t in range(1, L):
                acc = acc + buf[t * CHUNK_SEQS + i]
            gbase = (i // LANES) * LANES
            cnt16 = cnt_v[pl.ds(gbase, LANES)]
            sel = jnp.where(lane_iota == i - gbase, 1.0, 0.0)
            za = jnp.full((LANES,), jnp.sum(cnt16 * sel.astype(jnp.float32)))
            inv = 1.0 / jnp.maximum(lf - za, 1.0)
            outbuf[chunk * CHUNK_SEQS + i] = (acc - za * t0) * inv
            return carry

        lax.fori_loop(0, CHUNK_SEQS, seq_body, 0, unroll=False)

        if chunk + 2 < N_CHUNKS:
            inflight[chunk % 2] = fire(chunk + 2)

    pltpu.sync_copy(
        outbuf,
        out_hbm.at[pl.ds((wid % 4) * SEQ_PER_W, SEQ_PER_W),
                   pl.ds((wid // 4) * DIM, DIM)])


SLOT_BITS = 17
SLOT_W = 1 << SLOT_BITS
SLOTS = 8
TBLK = 8192
RGRID = SLOT_W // TBLK
_LAST_IN_BLK = (VOCAB + TBLK - 1) // TBLK - 1


def _repack_body(*refs):
    o_ref = refs[SLOTS]
    g = jnp.concatenate([r[...] for r in refs[:SLOTS]], axis=0)
    o_ref[...] = g.T


def _tc_repack(table_t):
    def mk(s):
        return pl.BlockSpec(
            (16, TBLK), lambda i, s=s: (0, jnp.minimum(s * RGRID + i,
                                                       _LAST_IN_BLK)))
    return pl.pallas_call(
        _repack_body,
        grid=(RGRID,),
        in_specs=[mk(s) for s in range(SLOTS)],
        out_specs=pl.BlockSpec((TBLK, 128), lambda i: (i, 0)),
        out_shape=jax.ShapeDtypeStruct((SLOT_W, 128), jnp.float32),
    )(*([table_t] * SLOTS))


def _unpack_body(x_ref, o_ref):
    y = x_ref[...].T
    for g in range(8):
        o_ref[:, g * XQ:(g + 1) * XQ] = y[g * DIM:(g + 1) * DIM, :]


def _tc_unpack(x):
    return pl.pallas_call(
        _unpack_body,
        out_shape=jax.ShapeDtypeStruct((DIM, B), jnp.float32),
    )(x)


@jax.jit
def _run(seq_cm, table):
    mesh = plsc.VectorSubcoreMesh(core_axis_name="c", subcore_axis_name="s")
    k = functools.partial(
        pl.kernel,
        mesh=mesh,
        out_type=jax.ShapeDtypeStruct((XQ, 128), jnp.float32),
        compiler_params=pltpu.CompilerParams(
            needs_layout_passes=False, use_tc_tiling_on_sc=False),
        scratch_types=[
            pltpu.VMEM((IDX_PER_W,), jnp.int32),
            pltpu.VMEM((CHUNK_IDX, DIM), jnp.float32),
            pltpu.VMEM((CHUNK_IDX, DIM), jnp.float32),
            pltpu.VMEM((SEQ_PER_W, DIM), jnp.float32),
            pltpu.VMEM((CHUNK_SEQS,), jnp.float32),
            pltpu.VMEM((1, DIM), jnp.float32),
            pltpu.SemaphoreType.DMA,
            pltpu.SemaphoreType.DMA,
            pltpu.SemaphoreType.DMA,
        ],
    )(_body)
    table_rm = _tc_repack(table.T).reshape(SLOTS * SLOT_W, DIM)
    return _tc_unpack(k(seq_cm, table_rm)).T


def kernel(sequences, table):
    return _run(sequences.T.reshape(B * L), table)


# speedup vs baseline: 6.6000x; 1.0223x over previous
import functools

import jax
import jax.numpy as jnp
from jax import lax
from jax.experimental import pallas as pl
from jax.experimental.pallas import tpu as pltpu
from jax.experimental.pallas import tpu_sc as plsc

VOCAB = 1000000
DIM = 16
B = 16384
L = 20
LANES = 16

NC = 2
NS = 16
NW = NC * NS

SEQ_PER_W = B // NW
IDX_PER_W = SEQ_PER_W * L
CHUNK_SEQS = 64
CHUNK_IDX = CHUNK_SEQS * L
N_CHUNKS = SEQ_PER_W // CHUNK_SEQS

XQ = B // 8


def _body(seq_cm_hbm, table_hbm, out_hbm,
          idx_flat, rows0, rows1, outbuf, cnt_v, t0_v,
          sem0, sem1, sem2):
    wid = lax.axis_index("s") * NC + lax.axis_index("c")
    sbase = wid * SEQ_PER_W

    stage = []
    for t in range(L):
        stage.append(pltpu.async_copy(
            seq_cm_hbm.at[pl.ds(t * B + sbase, SEQ_PER_W)],
            idx_flat.at[pl.ds(t * SEQ_PER_W, SEQ_PER_W)], sem2))
    pltpu.sync_copy(table_hbm.at[pl.ds(0, 1)], t0_v)
    for cp in stage:
        cp.wait()

    def xform(i, carry):
        v = idx_flat[pl.ds(i * LANES, LANES)]
        idx_flat[pl.ds(i * LANES, LANES)] = (
            ((v & (SLOT_W - 1)) << 3) | (v >> SLOT_BITS))
        return carry

    lax.fori_loop(0, IDX_PER_W // LANES, xform, 0, unroll=False)

    rows_bufs = (rows0, rows1)
    sems = (sem0, sem1)

    def fire(chunk):
        buf = rows_bufs[chunk % 2]
        sem = sems[chunk % 2]
        cps = []
        for t in range(L):
            idx_row = idx_flat.at[
                pl.ds(t * SEQ_PER_W + chunk * CHUNK_SEQS, CHUNK_SEQS)]
            dst = buf.at[pl.ds(t * CHUNK_SEQS, CHUNK_SEQS)]
            cps.append(pltpu.async_copy(table_hbm.at[idx_row], dst, sem))
        return cps

    inflight = [fire(0), fire(1)]

    t0 = t0_v[0]
    lf = jnp.float32(L)
    lane_iota = lax.iota(jnp.int32, LANES)

    for chunk in range(N_CHUNKS):
        buf = rows_bufs[chunk % 2]
        for cp in inflight[chunk % 2]:
            cp.wait()

        for g in range(CHUNK_SEQS // LANES):
            cnt = jnp.zeros((LANES,), jnp.float32)
            for t in range(L):
                v = idx_flat[pl.ds(
                    t * SEQ_PER_W + chunk * CHUNK_SEQS + g * LANES, LANES)]
                cnt = cnt + jnp.where(v == 0, 1.0, 0.0).astype(jnp.float32)
            cnt_v[pl.ds(g * LANES, LANES)] = cnt

        def seq_body(i, carry, chunk=chunk, buf=buf):
            acc = buf[i]
            for t in range(1, L):
                acc = acc + buf[t * CHUNK_SEQS + i]
            gbase = (i // LANES) * LANES
            cnt16 = cnt_v[pl.ds(gbase, LANES)]
            sel = jnp.where(lane_iota == i - gbase, 1.0, 0.0)
            za = jnp.full((LANES,), jnp.sum(cnt16 * sel.astype(jnp.float32)))
            inv = 1.0 / jnp.maximum(lf - za, 1.0)
            outbuf[chunk * CHUNK_SEQS + i] = (acc - za * t0) * inv
            return carry

        lax.fori_loop(0, CHUNK_SEQS, seq_body, 0, unroll=False)

        if chunk + 2 < N_CHUNKS:
            inflight[chunk % 2] = fire(chunk + 2)

    pltpu.sync_copy(
        outbuf,
        out_hbm.at[pl.ds((wid % 4) * SEQ_PER_W, SEQ_PER_W),
                   pl.ds((wid // 4) * DIM, DIM)])


SLOT_BITS = 17
SLOT_W = 1 << SLOT_BITS
SLOTS = 8
TBLK = 16384
RGRID = SLOT_W // TBLK
_LAST_IN_BLK = (VOCAB + TBLK - 1) // TBLK - 1


def _repack_body(*refs):
    o_ref = refs[SLOTS]
    g = jnp.concatenate([r[...] for r in refs[:SLOTS]], axis=0)
    o_ref[...] = g.T


def _tc_repack(table_t):
    def mk(s):
        return pl.BlockSpec(
            (16, TBLK), lambda i, s=s: (0, jnp.minimum(s * RGRID + i,
                                                       _LAST_IN_BLK)))
    return pl.pallas_call(
        _repack_body,
        grid=(RGRID,),
        in_specs=[mk(s) for s in range(SLOTS)],
        out_specs=pl.BlockSpec((TBLK, 128), lambda i: (i, 0)),
        out_shape=jax.ShapeDtypeStruct((SLOT_W, 128), jnp.float32),
    )(*([table_t] * SLOTS))


def _unpack_body(x_ref, o_ref):
    y = x_ref[...].T
    for g in range(8):
        o_ref[:, g * XQ:(g + 1) * XQ] = y[g * DIM:(g + 1) * DIM, :]


def _tc_unpack(x):
    return pl.pallas_call(
        _unpack_body,
        out_shape=jax.ShapeDtypeStruct((DIM, B), jnp.float32),
    )(x)


@jax.jit
def _run(seq_cm, table):
    mesh = plsc.VectorSubcoreMesh(core_axis_name="c", subcore_axis_name="s")
    k = functools.partial(
        pl.kernel,
        mesh=mesh,
        out_type=jax.ShapeDtypeStruct((XQ, 128), jnp.float32),
        compiler_params=pltpu.CompilerParams(
            needs_layout_passes=False, use_tc_tiling_on_sc=False),
        scratch_types=[
            pltpu.VMEM((IDX_PER_W,), jnp.int32),
            pltpu.VMEM((CHUNK_IDX, DIM), jnp.float32),
            pltpu.VMEM((CHUNK_IDX, DIM), jnp.float32),
            pltpu.VMEM((SEQ_PER_W, DIM), jnp.float32),
            pltpu.VMEM((CHUNK_SEQS,), jnp.float32),
            pltpu.VMEM((1, DIM), jnp.float32),
            pltpu.SemaphoreType.DMA,
            pltpu.SemaphoreType.DMA,
            pltpu.SemaphoreType.DMA,
        ],
    )(_body)
    table_rm = _tc_repack(table.T).reshape(SLOTS * SLOT_W, DIM)
    return _tc_unpack(k(seq_cm, table_rm)).T


def kernel(sequences, table):
    return _run(sequences.T.reshape(B * L), table)


# speedup vs baseline: 6.7467x; 1.0222x over previous
import functools

import jax
import jax.numpy as jnp
from jax import lax
from jax.experimental import pallas as pl
from jax.experimental.pallas import tpu as pltpu
from jax.experimental.pallas import tpu_sc as plsc

VOCAB = 1000000
DIM = 16
B = 16384
L = 20
LANES = 16

NC = 2
NS = 16
NW = NC * NS

SEQ_PER_W = B // NW
IDX_PER_W = SEQ_PER_W * L
CHUNK_SEQS = 64
CHUNK_IDX = CHUNK_SEQS * L
N_CHUNKS = SEQ_PER_W // CHUNK_SEQS

XQ = B // 8


def _idx_body(seq_cm_hbm, ridx_hbm, idx_flat, sem):
    wid = lax.axis_index("s") * NC + lax.axis_index("c")
    sbase = wid * SEQ_PER_W

    stage = []
    for t in range(L):
        stage.append(pltpu.async_copy(
            seq_cm_hbm.at[pl.ds(t * B + sbase, SEQ_PER_W)],
            idx_flat.at[pl.ds(t * SEQ_PER_W, SEQ_PER_W)], sem))
    for cp in stage:
        cp.wait()

    def xform(i, carry):
        v = idx_flat[pl.ds(i * LANES, LANES)]
        idx_flat[pl.ds(i * LANES, LANES)] = (
            ((v & (SLOT_W - 1)) << 3) | (v >> SLOT_BITS))
        return carry

    lax.fori_loop(0, IDX_PER_W // LANES, xform, 0, unroll=False)

    pltpu.sync_copy(idx_flat,
                    ridx_hbm.at[pl.ds(wid * IDX_PER_W, IDX_PER_W)])


def _body(ridx_hbm, table_hbm, out_hbm,
          idx_flat, rows0, rows1, outbuf, cnt_v, t0_v,
          sem0, sem1, sem2):
    wid = lax.axis_index("s") * NC + lax.axis_index("c")

    cp0 = pltpu.async_copy(ridx_hbm.at[pl.ds(wid * IDX_PER_W, IDX_PER_W)],
                           idx_flat.at[pl.ds(0, IDX_PER_W)], sem2)
    pltpu.sync_copy(table_hbm.at[pl.ds(0, 1)], t0_v)
    cp0.wait()

    rows_bufs = (rows0, rows1)
    sems = (sem0, sem1)

    def fire(chunk):
        buf = rows_bufs[chunk % 2]
        sem = sems[chunk % 2]
        cps = []
        for t in range(L):
            idx_row = idx_flat.at[
                pl.ds(t * SEQ_PER_W + chunk * CHUNK_SEQS, CHUNK_SEQS)]
            dst = buf.at[pl.ds(t * CHUNK_SEQS, CHUNK_SEQS)]
            cps.append(pltpu.async_copy(table_hbm.at[idx_row], dst, sem))
        return cps

    inflight = [fire(0), fire(1)]

    t0 = t0_v[0]
    lf = jnp.float32(L)
    lane_iota = lax.iota(jnp.int32, LANES)

    for chunk in range(N_CHUNKS):
        buf = rows_bufs[chunk % 2]
        for cp in inflight[chunk % 2]:
            cp.wait()

        for g in range(CHUNK_SEQS // LANES):
            cnt = jnp.zeros((LANES,), jnp.float32)
            for t in range(L):
                v = idx_flat[pl.ds(
                    t * SEQ_PER_W + chunk * CHUNK_SEQS + g * LANES, LANES)]
                cnt = cnt + jnp.where(v == 0, 1.0, 0.0).astype(jnp.float32)
            cnt_v[pl.ds(g * LANES, LANES)] = cnt

        def seq_body(i, carry, chunk=chunk, buf=buf):
            acc = buf[i]
            for t in range(1, L):
                acc = acc + buf[t * CHUNK_SEQS + i]
            gbase = (i // LANES) * LANES
            cnt16 = cnt_v[pl.ds(gbase, LANES)]
            sel = jnp.where(lane_iota == i - gbase, 1.0, 0.0)
            za = jnp.full((LANES,), jnp.sum(cnt16 * sel.astype(jnp.float32)))
            inv = 1.0 / jnp.maximum(lf - za, 1.0)
            outbuf[chunk * CHUNK_SEQS + i] = (acc - za * t0) * inv
            return carry

        lax.fori_loop(0, CHUNK_SEQS, seq_body, 0, unroll=False)

        if chunk + 2 < N_CHUNKS:
            inflight[chunk % 2] = fire(chunk + 2)

    pltpu.sync_copy(
        outbuf,
        out_hbm.at[pl.ds((wid % 4) * SEQ_PER_W, SEQ_PER_W),
                   pl.ds((wid // 4) * DIM, DIM)])


SLOT_BITS = 17
SLOT_W = 1 << SLOT_BITS
SLOTS = 8
TBLK = 16384
RGRID = SLOT_W // TBLK
_LAST_IN_BLK = (VOCAB + TBLK - 1) // TBLK - 1


def _repack_body(*refs):
    o_ref = refs[SLOTS]
    g = jnp.concatenate([r[...] for r in refs[:SLOTS]], axis=0)
    o_ref[...] = g.T


def _tc_repack(table_t):
    def mk(s):
        return pl.BlockSpec(
            (16, TBLK), lambda i, s=s: (0, jnp.minimum(s * RGRID + i,
                                                       _LAST_IN_BLK)))
    return pl.pallas_call(
        _repack_body,
        grid=(RGRID,),
        in_specs=[mk(s) for s in range(SLOTS)],
        out_specs=pl.BlockSpec((TBLK, 128), lambda i: (i, 0)),
        out_shape=jax.ShapeDtypeStruct((SLOT_W, 128), jnp.float32),
    )(*([table_t] * SLOTS))


def _unpack_body(x_ref, o_ref):
    y = x_ref[...].T
    for g in range(8):
        o_ref[:, g * XQ:(g + 1) * XQ] = y[g * DIM:(g + 1) * DIM, :]


def _tc_unpack(x):
    return pl.pallas_call(
        _unpack_body,
        out_shape=jax.ShapeDtypeStruct((DIM, B), jnp.float32),
    )(x)


@jax.jit
def _run(seq_cm, table):
    mesh = plsc.VectorSubcoreMesh(core_axis_name="c", subcore_axis_name="s")
    k_idx = functools.partial(
        pl.kernel,
        mesh=mesh,
        out_type=jax.ShapeDtypeStruct((B * L,), jnp.int32),
        compiler_params=pltpu.CompilerParams(
            needs_layout_passes=False, use_tc_tiling_on_sc=False),
        scratch_types=[
            pltpu.VMEM((IDX_PER_W,), jnp.int32),
            pltpu.SemaphoreType.DMA,
        ],
    )(_idx_body)
    k = functools.partial(
        pl.kernel,
        mesh=mesh,
        out_type=jax.ShapeDtypeStruct((XQ, 128), jnp.float32),
        compiler_params=pltpu.CompilerParams(
            needs_layout_passes=False, use_tc_tiling_on_sc=False),
        scratch_types=[
            pltpu.VMEM((IDX_PER_W,), jnp.int32),
            pltpu.VMEM((CHUNK_IDX, DIM), jnp.float32),
            pltpu.VMEM((CHUNK_IDX, DIM), jnp.float32),
            pltpu.VMEM((SEQ_PER_W, DIM), jnp.float32),
            pltpu.VMEM((CHUNK_SEQS,), jnp.float32),
            pltpu.VMEM((1, DIM), jnp.float32),
            pltpu.SemaphoreType.DMA,
            pltpu.SemaphoreType.DMA,
            pltpu.SemaphoreType.DMA,
        ],
    )(_body)
    ridx = k_idx(seq_cm)
    table_rm = _tc_repack(table.T).reshape(SLOTS * SLOT_W, DIM)
    return _tc_unpack(k(ridx, table_rm)).T


def kernel(sequences, table):
    return _run(sequences.T.reshape(B * L), table)
